# baseline (speedup 1.0000x reference)
import functools

import jax
import jax.numpy as jnp
from jax.experimental import pallas as pl
from jax.experimental.pallas import tpu as pltpu

_MAX_SPAN_LEN = 16
_WIN = 24


def _pool_body(pk_sm, nsp_sm, sn_sm, emb_hbm, sv_ref, ev_ref,
               out_ref, mask_ref, gbuf, sem16, sem8, *, S, D):
    b = pl.program_id(0)
    nb = pl.num_programs(0)
    slot = jax.lax.rem(b, 8)

    def issue(bb, sl):
        for mi in range(S):
            v = pk_sm[bb * S + mi]
            base = pl.multiple_of(v & 0xFFFF, 8)
            pltpu.make_async_copy(
                emb_hbm.at[bb, pl.ds(base, 16), :],
                gbuf.at[sl, pl.ds(mi * _WIN, 16), :],
                sem16.at[sl],
            ).start()

            @pl.when((v >> 16) != 0)
            def _():
                pltpu.make_async_copy(
                    emb_hbm.at[bb, pl.ds(base + 16, 8), :],
                    gbuf.at[sl, pl.ds(mi * _WIN + 16, 8), :],
                    sem8.at[sl],
                ).start()

    @pl.when(b == 0)
    def _():
        issue(0, 0)
        issue(1, 1)
        issue(2, 2)
        issue(3, 3)

    @pl.when(b + 4 < nb)
    def _():
        issue(b + 4, jax.lax.rem(b + 4, 8))

    sn = sn_sm[b]
    span_iota = jax.lax.broadcasted_iota(jnp.int32, (1, S), 1)
    valid_span = span_iota < sn
    mask_ref[0] = valid_span.astype(jnp.int32)

    sv = sv_ref[0]
    ev = ev_ref[0]
    lo = sv - ((sv >> 3) << 3)
    cnt = jnp.minimum(ev - sv, _MAX_SPAN_LEN)
    hi = lo + cnt
    inv = 1.0 / jnp.maximum(cnt, 1).astype(jnp.float32)
    scale = jnp.where(valid_span & (cnt > 0), inv, 0.0)

    k_iota = jax.lax.broadcasted_iota(jnp.int32, (S * _WIN, S), 0)
    mi_iota = jax.lax.broadcasted_iota(jnp.int32, (S * _WIN, S), 1)
    off = k_iota - mi_iota * _WIN
    wt = jnp.where((off >= lo) & (off < hi), scale, 0.0)

    pltpu.make_async_copy(
        emb_hbm.at[b, pl.ds(0, S * 16), :],
        gbuf.at[slot, pl.ds(0, S * 16), :],
        sem16.at[slot],
    ).wait()
    ns = nsp_sm[b]

    @pl.when(ns > 0)
    def _():
        pltpu.make_async_copy(
            emb_hbm.at[b, pl.ds(0, 8 * ns), :],
            gbuf.at[slot, pl.ds(0, 8 * ns), :],
            sem8.at[slot],
        ).wait()

    out_ref[0] = jax.lax.dot_general(
        wt, gbuf[slot], (((0,), (0,)), ((), ())),
        preferred_element_type=jnp.float32)


def kernel(embeddings, span_positions, span_nums, repeated_vector):
    B, L, D = embeddings.shape
    S = span_positions.shape[1]
    sp = span_positions.astype(jnp.int32) + 1
    starts = sp[..., 0]
    ends = sp[..., 1]
    lo = starts & 7
    cnt = jnp.clip(ends - starts, 0, _MAX_SPAN_LEN)
    base_rows = (starts >> 3) * 8
    spill = (lo + jnp.maximum(cnt, 1)) > 16
    spill = spill | (jnp.arange(B, dtype=jnp.int32)[:, None] <= 7)
    pk = (base_rows | (spill.astype(jnp.int32) << 16)).reshape(-1)
    n_spill = spill.astype(jnp.int32).sum(axis=1)
    sn = span_nums.astype(jnp.int32)
    sv = starts.reshape(B, 1, S)
    ev = ends.reshape(B, 1, S)

    body = functools.partial(_pool_body, S=S, D=D)
    grid_spec = pltpu.PrefetchScalarGridSpec(
        num_scalar_prefetch=3,
        grid=(B,),
        in_specs=[pl.BlockSpec(memory_space=pl.ANY),
                  pl.BlockSpec((1, 1, S), lambda b, *_: (b, 0, 0)),
                  pl.BlockSpec((1, 1, S), lambda b, *_: (b, 0, 0))],
        out_specs=[pl.BlockSpec((1, S, D), lambda b, *_: (b, 0, 0)),
                   pl.BlockSpec((1, 1, S), lambda b, *_: (b, 0, 0))],
        scratch_shapes=[
            pltpu.VMEM((8, S * _WIN, D), jnp.float32),
            pltpu.SemaphoreType.DMA((8,)),
            pltpu.SemaphoreType.DMA((8,)),
        ],
    )
    pooled, maski = pl.pallas_call(
        body,
        grid_spec=grid_spec,
        out_shape=[jax.ShapeDtypeStruct((B, S, D), jnp.float32),
                   jax.ShapeDtypeStruct((B, 1, S), jnp.int32)],
        compiler_params=pltpu.CompilerParams(
            dimension_semantics=("arbitrary",),
        ),
        name="span_mean_pool_dma_mxu",
    )(pk, n_spill, sn, embeddings, sv, ev)
    return pooled, maski.reshape(B, S) > 0


# speedup vs baseline: 1.0104x; 1.0104x over previous
import functools

import jax
import jax.numpy as jnp
from jax.experimental import pallas as pl
from jax.experimental.pallas import tpu as pltpu

_MAX_SPAN_LEN = 16
_WIN = 24


def _pool_body(pk_sm, nsp_sm, sn_sm, emb_hbm, sv_ref, ev_ref,
               out_ref, mask_ref, gbuf, sem16, sem8, *, S, D):
    b = pl.program_id(0)
    nb = pl.num_programs(0)
    slot = jax.lax.rem(b, 6)

    def issue(bb, sl):
        for mi in range(S):
            v = pk_sm[bb * S + mi]
            base = pl.multiple_of(v & 0xFFFF, 8)
            pltpu.make_async_copy(
                emb_hbm.at[bb, pl.ds(base, 16), :],
                gbuf.at[sl, pl.ds(mi * _WIN, 16), :],
                sem16.at[sl],
            ).start()

            @pl.when((v >> 16) != 0)
            def _():
                pltpu.make_async_copy(
                    emb_hbm.at[bb, pl.ds(base + 16, 8), :],
                    gbuf.at[sl, pl.ds(mi * _WIN + 16, 8), :],
                    sem8.at[sl],
                ).start()

    @pl.when(b == 0)
    def _():
        issue(0, 0)
        issue(1, 1)
        issue(2, 2)

    @pl.when(b + 3 < nb)
    def _():
        issue(b + 3, jax.lax.rem(b + 3, 6))

    sn = sn_sm[b]
    span_iota = jax.lax.broadcasted_iota(jnp.int32, (1, S), 1)
    valid_span = span_iota < sn
    mask_ref[0] = valid_span.astype(jnp.int32)

    sv = sv_ref[0]
    ev = ev_ref[0]
    lo = sv - ((sv >> 3) << 3)
    cnt = jnp.minimum(ev - sv, _MAX_SPAN_LEN)
    hi = lo + cnt
    inv = 1.0 / jnp.maximum(cnt, 1).astype(jnp.float32)
    scale = jnp.where(valid_span & (cnt > 0), inv, 0.0)

    k_iota = jax.lax.broadcasted_iota(jnp.int32, (S * _WIN, S), 0)
    mi_iota = jax.lax.broadcasted_iota(jnp.int32, (S * _WIN, S), 1)
    off = k_iota - mi_iota * _WIN
    wt = jnp.where((off >= lo) & (off < hi), scale, 0.0)

    pltpu.make_async_copy(
        emb_hbm.at[b, pl.ds(0, S * 16), :],
        gbuf.at[slot, pl.ds(0, S * 16), :],
        sem16.at[slot],
    ).wait()
    ns = nsp_sm[b]

    @pl.when(ns > 0)
    def _():
        pltpu.make_async_copy(
            emb_hbm.at[b, pl.ds(0, 8 * ns), :],
            gbuf.at[slot, pl.ds(0, 8 * ns), :],
            sem8.at[slot],
        ).wait()

    out_ref[0] = jax.lax.dot_general(
        wt, gbuf[slot], (((0,), (0,)), ((), ())),
        preferred_element_type=jnp.float32)


def kernel(embeddings, span_positions, span_nums, repeated_vector):
    B, L, D = embeddings.shape
    S = span_positions.shape[1]
    sp = span_positions.astype(jnp.int32) + 1
    starts = sp[..., 0]
    ends = sp[..., 1]
    lo = starts & 7
    cnt = jnp.clip(ends - starts, 0, _MAX_SPAN_LEN)
    base_rows = (starts >> 3) * 8
    spill = (lo + jnp.maximum(cnt, 1)) > 16
    spill = spill | (jnp.arange(B, dtype=jnp.int32)[:, None] <= 5)
    pk = (base_rows | (spill.astype(jnp.int32) << 16)).reshape(-1)
    n_spill = spill.astype(jnp.int32).sum(axis=1)
    sn = span_nums.astype(jnp.int32)
    sv = starts.reshape(B, 1, S)
    ev = ends.reshape(B, 1, S)

    body = functools.partial(_pool_body, S=S, D=D)
    grid_spec = pltpu.PrefetchScalarGridSpec(
        num_scalar_prefetch=3,
        grid=(B,),
        in_specs=[pl.BlockSpec(memory_space=pl.ANY),
                  pl.BlockSpec((1, 1, S), lambda b, *_: (b, 0, 0)),
                  pl.BlockSpec((1, 1, S), lambda b, *_: (b, 0, 0))],
        out_specs=[pl.BlockSpec((1, S, D), lambda b, *_: (b, 0, 0)),
                   pl.BlockSpec((1, 1, S), lambda b, *_: (b, 0, 0))],
        scratch_shapes=[
            pltpu.VMEM((6, S * _WIN, D), jnp.float32),
            pltpu.SemaphoreType.DMA((6,)),
            pltpu.SemaphoreType.DMA((6,)),
        ],
    )
    pooled, maski = pl.pallas_call(
        body,
        grid_spec=grid_spec,
        out_shape=[jax.ShapeDtypeStruct((B, S, D), jnp.float32),
                   jax.ShapeDtypeStruct((B, 1, S), jnp.int32)],
        compiler_params=pltpu.CompilerParams(
            dimension_semantics=("arbitrary",),
        ),
        name="span_mean_pool_dma_mxu",
    )(pk, n_spill, sn, embeddings, sv, ev)
    return pooled, maski.reshape(B, S) > 0
